# baseline (speedup 1.0000x reference)
import jax
import jax.numpy as jnp
from jax import lax
from jax.experimental import pallas as pl
from jax.experimental.pallas import tpu as pltpu
from jax.experimental.pallas import tpu_sc as plsc

N = 10000
E = 320000
NG = 16
L = 4
EPS = 1e-5
NEG = -1e30

NC = 2
NS = 16
LANES = 16
CE = 128
CPT = 160
CPB = 16
EPT = CPT * CE
E_PAD = NS * EPT
HALF = 5120
HROWS = 6144
RPT = HROWS // NS
DPAD = 99999
BE = 2048


def _make_edge_pass(with_ae, with_deg=False):
    feat = 128
    mesh = plsc.VectorSubcoreMesh(core_axis_name="c", subcore_axis_name="s")
    scratch = [
        pltpu.VMEM((CPB, CE), jnp.int32),
        pltpu.VMEM((CPB, CE), jnp.int32),
        pltpu.VMEM((CE,), jnp.int32),
        pltpu.VMEM((CE, feat), jnp.float32),
        pltpu.VMEM((CE, feat), jnp.float32),
        [pltpu.SemaphoreType.DMA, pltpu.SemaphoreType.DMA],
        pltpu.VMEM_SHARED((HROWS, feat), jnp.float32),
    ]
    out_type = [jax.ShapeDtypeStruct((NC, HROWS, feat), jnp.float32)]
    if with_deg:
        scratch.append(pltpu.VMEM((CE, LANES), jnp.float32))
        scratch.append(
            pltpu.VMEM_SHARED((HROWS, LANES), jnp.float32))
        out_type.append(
            jax.ShapeDtypeStruct((NC, HROWS, LANES), jnp.float32))

    def body(table, ae, src, dst, *rest):
        if with_deg:
            (out, dout, src_v, dst_v, idx_v, rows_v, ae_v, semg, acc_sh,
             ones_v, dacc_sh) = rest
        else:
            (out, src_v, dst_v, idx_v, rows_v, ae_v, semg, acc_sh) = rest
        cid = lax.axis_index("c")
        sid = lax.axis_index("s")

        zeros = jnp.zeros((LANES,), jnp.float32)
        ones = jnp.full((LANES,), 1.0, jnp.float32)

        def zrow(i, _):
            for j in range(feat // LANES):
                rows_v[i, pl.ds(j * LANES, LANES)] = zeros
            if with_deg:
                ones_v[i, pl.ds(0, LANES)] = zeros
            return 0

        lax.fori_loop(0, CE, zrow, 0)
        for k in range(RPT // CE):
            pltpu.sync_copy(rows_v,
                            acc_sh.at[pl.ds(sid * RPT + k * CE, CE)])
            if with_deg:
                pltpu.sync_copy(
                    ones_v, dacc_sh.at[pl.ds(sid * RPT + k * CE, CE)])
        if with_deg:
            def onerow(i, _):
                ones_v[i, pl.ds(0, LANES)] = ones
                return 0

            lax.fori_loop(0, CE, onerow, 0)
        plsc.subcore_barrier()

        base = cid * HALF

        def chunk(k, nb):
            c = nb * CPB + k
            g = pltpu.async_copy(table.at[src_v.at[k]], rows_v, semg[0])
            if with_ae:
                pltpu.async_copy(
                    ae.at[pl.ds((sid * CPT + c) * CE, CE)], ae_v, semg[1])
            for j in range(CE // LANES):
                sl = pl.ds(j * LANES, LANES)
                d = dst_v[k, sl] - base
                ok = (d >= 0) & (d < HALF)
                idx_v[sl] = jnp.where(ok, d, HALF)
            if with_deg:
                pltpu.sync_copy(ones_v, dacc_sh.at[idx_v], add=True)
            g.wait()
            if with_ae:
                pltpu.make_async_copy(
                    ae.at[pl.ds((sid * CPT + c) * CE, CE)], ae_v,
                    semg[1]).wait()

                def erow(i, _):
                    for j in range(feat // LANES):
                        sl = pl.ds(j * LANES, LANES)
                        rows_v[i, sl] = jnp.maximum(
                            rows_v[i, sl] + ae_v[i, sl], 0.0)
                    return 0

                lax.fori_loop(0, CE, erow, 0)
            pltpu.sync_copy(rows_v, acc_sh.at[idx_v], add=True)
            return nb

        def block(nb, _):
            pltpu.sync_copy(src.at[sid, pl.ds(nb * CPB, CPB)], src_v)
            pltpu.sync_copy(dst.at[sid, pl.ds(nb * CPB, CPB)], dst_v)
            lax.fori_loop(0, CPB, chunk, nb)
            return 0

        lax.fori_loop(0, CPT // CPB, block, 0)
        plsc.subcore_barrier()

        for k in range(RPT // CE):
            r0 = sid * RPT + k * CE
            pltpu.sync_copy(acc_sh.at[pl.ds(r0, CE)], rows_v)
            pltpu.sync_copy(rows_v, out.at[cid, pl.ds(r0, CE)])
        if with_deg:
            for k in range(RPT // CE):
                r0 = sid * RPT + k * CE
                pltpu.sync_copy(dacc_sh.at[pl.ds(r0, CE)], ones_v)
                pltpu.sync_copy(ones_v, dout.at[cid, pl.ds(r0, CE)])

    return pl.kernel(
        body,
        out_type=tuple(out_type) if with_deg else out_type[0],
        mesh=mesh,
        scratch_types=scratch,
    )


def _make_deg_pass():
    mesh = plsc.VectorSubcoreMesh(core_axis_name="c", subcore_axis_name="s")
    scratch = [
        pltpu.VMEM((CPT, CE), jnp.int32),
        pltpu.VMEM((CE,), jnp.int32),
        pltpu.VMEM((CE, LANES), jnp.float32),
        pltpu.VMEM_SHARED((HROWS, LANES), jnp.float32),
    ]

    def body(dst, out, dst_v, idx_v, buf_v, acc_sh):
        cid = lax.axis_index("c")
        sid = lax.axis_index("s")
        pltpu.sync_copy(dst.at[sid], dst_v)

        zeros = jnp.zeros((LANES,), jnp.float32)

        def zrow(i, _):
            buf_v[i, pl.ds(0, LANES)] = zeros
            return 0

        lax.fori_loop(0, CE, zrow, 0)
        for k in range(RPT // CE):
            pltpu.sync_copy(buf_v, acc_sh.at[pl.ds(sid * RPT + k * CE, CE)])
        plsc.subcore_barrier()

        ones = jnp.full((LANES,), 1.0, jnp.float32)

        def orow(i, _):
            buf_v[i, pl.ds(0, LANES)] = ones
            return 0

        lax.fori_loop(0, CE, orow, 0)
        base = cid * HALF

        def chunk(c, _):
            for j in range(CE // LANES):
                sl = pl.ds(j * LANES, LANES)
                d = dst_v[c, sl] - base
                ok = (d >= 0) & (d < HALF)
                idx_v[sl] = jnp.where(ok, d, HALF)
            pltpu.sync_copy(buf_v, acc_sh.at[idx_v], add=True)
            return 0

        lax.fori_loop(0, CPT, chunk, 0)
        plsc.subcore_barrier()

        for k in range(RPT // CE):
            r0 = sid * RPT + k * CE
            pltpu.sync_copy(acc_sh.at[pl.ds(r0, CE)], buf_v)
            pltpu.sync_copy(buf_v, out.at[cid, pl.ds(r0, CE)])

    return pl.kernel(
        body,
        out_type=jax.ShapeDtypeStruct((NC, HROWS, LANES), jnp.float32),
        mesh=mesh,
        scratch_types=scratch,
    )


def _contract0(a, b):
    return lax.dot_general(a, b, (((0,), (0,)), ((), ())))


def _graph_norm(x, oh, gamma, beta, alpha):
    ones = jnp.ones((x.shape[0], 1), jnp.float32)
    cnt = jnp.maximum(_contract0(oh, ones), 1.0)
    mean = _contract0(oh, x) / cnt
    xc = x - alpha * jnp.dot(oh, mean)
    var = _contract0(oh, xc * xc) / cnt
    vb = jnp.dot(oh, var)
    return gamma * xc * lax.rsqrt(vb + EPS) + beta


def _onehot(batch_col):
    g = lax.broadcasted_iota(jnp.int32, (N, NG), 1)
    return (batch_col == g).astype(jnp.float32)


def _cat(a0, a1):
    return jnp.concatenate([a0, a1], axis=0)


def _encoder_body(point, cond, tim, batch, W1, b1, g1, be1, a1, W2, b2,
                  h_out):
    h1 = (jnp.dot(point[...], W1[0:3, :])
          + jnp.dot(cond[...], W1[3:7, :])
          + jnp.dot(tim[...], W1[7:8, :]) + b1[...])
    oh = _onehot(batch[...])
    y = _graph_norm(h1, oh, g1[...], be1[...], a1[...])
    h_out[...] = jnp.dot(jnp.maximum(y, 0.0), W2[...]) + b2[...]


def _layer0_body(h, Wn, hw_out):
    hw_out[...] = jnp.dot(h[...], Wn[...])


def _layer_body(h, a0, a1, Wn, h_out, hw_out):
    hn = jnp.maximum(h[...] + _cat(a0[...], a1[...]), 0.0)
    h_out[...] = hn
    hw_out[...] = jnp.dot(hn, Wn[...])


def _ae_body(attr, eeW, eeb, We, bl, ae_out):
    i = pl.program_id(0)
    at = jnp.dot(attr[...], eeW[...]) + eeb[...]
    ae = jnp.dot(at, We[...]) + bl[...]
    rows = i * BE + lax.broadcasted_iota(jnp.int32, (BE, 128), 0)
    ae_out[...] = jnp.where(rows < E, ae, NEG)


def _dec1_pre_body(h, a0, a1, d0, d1, Wd, y_out, dinv_out):
    hf = jnp.maximum(h[...] + _cat(a0[...], a1[...]), 0.0)
    dinv = lax.rsqrt(_cat(d0[...], d1[...]) + 1.0)
    dinv_out[...] = dinv
    y_out[...] = jnp.dot(hf, Wd[...]) * dinv


def _dec1_post_body(y, c0, c1, dinv, b1, batch, g1, be1, a1, Wd2, y2_out):
    dv = dinv[...]
    x = dv * (y[...] + _cat(c0[...], c1[...])) + b1[...]
    oh = _onehot(batch[...])
    x = jnp.maximum(_graph_norm(x, oh, g1[...], be1[...], a1[...]), 0.0)
    xw = jnp.dot(x, Wd2[...]) * dv
    y2_out[...] = jnp.concatenate(
        [xw, jnp.zeros((N, 64), jnp.float32)], axis=1)


def _final_body(y2, c0, c1, dinv, b2, batch, g2, be2, a2, fcW, fcb, out):
    cc = _cat(c0[...], c1[...])
    x = dinv[...] * (y2[:, 0:64] + cc[:, 0:64]) + b2[...]
    oh = _onehot(batch[...])
    x = jnp.maximum(_graph_norm(x, oh, g2[...], be2[...], a2[...]), 0.0)
    out[...] = jnp.dot(x, fcW[...]) + fcb[...]


def _tc_call(body, out_shapes):
    return pl.pallas_call(body, out_shape=out_shapes)


def kernel(point, condition, index, attr, batch, time, enc_W1, enc_b1,
           enc_g1, enc_be1, enc_a1, enc_W2, enc_b2, ee_W, ee_b, gcn_Wn,
           gcn_We, gcn_bl, dec_W1, dec_b1, dec_g1, dec_be1, dec_a1, dec_W2,
           dec_b2, dec_g2, dec_be2, dec_a2, fc_W, fc_b):
    f32 = jnp.float32
    src = index[0].astype(jnp.int32)
    dst = index[1].astype(jnp.int32)

    src_p = jnp.pad(src, (0, E_PAD - E)).reshape(NS, CPT, CE)
    dst_p = jnp.pad(dst, (0, E_PAD - E),
                    constant_values=DPAD).reshape(NS, CPT, CE)
    attr_p = jnp.pad(attr, ((0, E_PAD - E), (0, 0)))

    batch_c = batch.astype(jnp.int32).reshape(N, 1)
    row1 = lambda v: v.reshape(1, -1)

    edge_relu = _make_edge_pass(True)
    edge_gs = _make_edge_pass(False)
    deg_pass = _make_deg_pass()

    def parts(aggs):
        return aggs[0, :HALF], aggs[1, :N - HALF]

    h = _tc_call(_encoder_body, jax.ShapeDtypeStruct((N, 128), f32))(
        point, condition, time, batch_c, enc_W1, row1(enc_b1),
        row1(enc_g1), row1(enc_be1), row1(enc_a1), enc_W2, row1(enc_b2))

    degs = deg_pass(dst_p)
    d0 = degs[0, :HALF, 0:1]
    d1 = degs[1, :N - HALF, 0:1]

    ae_call = pl.pallas_call(
        _ae_body,
        grid=(E_PAD // BE,),
        in_specs=[
            pl.BlockSpec((BE, 16), lambda i: (i, 0)),
            pl.BlockSpec((16, 16), lambda i: (0, 0)),
            pl.BlockSpec((1, 16), lambda i: (0, 0)),
            pl.BlockSpec((16, 128), lambda i: (0, 0)),
            pl.BlockSpec((1, 128), lambda i: (0, 0)),
        ],
        out_specs=pl.BlockSpec((BE, 128), lambda i: (i, 0)),
        out_shape=jax.ShapeDtypeStruct((E_PAD, 128), f32),
    )

    agg = None
    for l in range(L):
        ae = ae_call(attr_p, ee_W, row1(ee_b), gcn_We[l], row1(gcn_bl[l]))
        if l == 0:
            hw = _tc_call(_layer0_body,
                          jax.ShapeDtypeStruct((N, 128), f32))(h, gcn_Wn[0])
        else:
            a0, a1 = parts(agg)
            h, hw = _tc_call(
                _layer_body,
                (jax.ShapeDtypeStruct((N, 128), f32),
                 jax.ShapeDtypeStruct((N, 128), f32)))(h, a0, a1, gcn_Wn[l])
        agg = edge_relu(hw, ae, src_p, dst_p)

    a0, a1 = parts(agg)
    y, dinv = _tc_call(
        _dec1_pre_body,
        (jax.ShapeDtypeStruct((N, 128), f32),
         jax.ShapeDtypeStruct((N, 1), f32)))(h, a0, a1, d0, d1, dec_W1)
    cagg = edge_gs(y, y, src_p, dst_p)

    c0, c1 = parts(cagg)
    y2 = _tc_call(_dec1_post_body, jax.ShapeDtypeStruct((N, 128), f32))(
        y, c0, c1, dinv, row1(dec_b1), batch_c, row1(dec_g1),
        row1(dec_be1), row1(dec_a1), dec_W2)
    cagg2 = edge_gs(y2, y2, src_p, dst_p)

    c0, c1 = parts(cagg2)
    out = _tc_call(_final_body, jax.ShapeDtypeStruct((N, 2), f32))(
        y2, c0, c1, dinv, row1(dec_b2), batch_c, row1(dec_g2),
        row1(dec_be2), row1(dec_a2), fc_W, row1(fc_b))
    return out


# speedup vs baseline: 1.5687x; 1.5687x over previous
import jax
import jax.numpy as jnp
from jax import lax
from jax.experimental import pallas as pl
from jax.experimental.pallas import tpu as pltpu
from jax.experimental.pallas import tpu_sc as plsc

N = 10000
E = 320000
NG = 16
L = 4
EPS = 1e-5
NEG = -1e30

NC = 2
NS = 16
LANES = 16
CE = 128
CPT = 157
EPT = CPT * CE
E_PAD = NS * EPT
HALF = 5120
HROWS = 6144
RPT = HROWS // NS
DPAD = 99999
BE = 2048


def _make_edge_pass(with_ae):
    feat = 128
    mesh = plsc.VectorSubcoreMesh(core_axis_name="c", subcore_axis_name="s")
    scratch = [
        pltpu.VMEM((CPT, CE), jnp.int32),
        pltpu.VMEM((CPT, CE), jnp.int32),
        pltpu.VMEM((CE,), jnp.int32),
        pltpu.VMEM((CE, feat), jnp.float32),
        pltpu.VMEM((CE, feat), jnp.float32),
        pltpu.VMEM_SHARED((HROWS, feat), jnp.float32),
        pltpu.SemaphoreType.DMA,
        pltpu.SemaphoreType.DMA,
    ]

    def body(table, ae, src, dst, out, src_v, dst_v, idx_v, rows_v, ae_v,
             acc_sh, sem, sem2):
        cid = lax.axis_index("c")
        sid = lax.axis_index("s")

        pltpu.sync_copy(src.at[sid], src_v)
        pltpu.sync_copy(dst.at[sid], dst_v)

        zeros = jnp.zeros((LANES,), jnp.float32)

        def zrow(i, _):
            for j in range(feat // LANES):
                rows_v[i, pl.ds(j * LANES, LANES)] = zeros
            return 0

        lax.fori_loop(0, CE, zrow, 0)
        for k in range(RPT // CE):
            pltpu.sync_copy(rows_v, acc_sh.at[pl.ds(sid * RPT + k * CE, CE)])
        plsc.subcore_barrier()

        base = cid * HALF

        def chunk(c, _):
            g = pltpu.async_copy(table.at[src_v.at[c]], rows_v, sem)
            if with_ae:
                a = pltpu.async_copy(
                    ae.at[pl.ds((sid * CPT + c) * CE, CE)], ae_v, sem2)
            for j in range(CE // LANES):
                sl = pl.ds(j * LANES, LANES)
                d = dst_v[c, sl] - base
                ok = (d >= 0) & (d < HALF)
                idx_v[sl] = jnp.where(ok, d, HALF)
            g.wait()
            if with_ae:
                a.wait()

                def erow(i, _):
                    for j in range(feat // LANES):
                        sl = pl.ds(j * LANES, LANES)
                        rows_v[i, sl] = jnp.maximum(
                            rows_v[i, sl] + ae_v[i, sl], 0.0)
                    return 0

                lax.fori_loop(0, CE, erow, 0)
            pltpu.sync_copy(rows_v, acc_sh.at[idx_v], add=True)
            return 0

        lax.fori_loop(0, CPT, chunk, 0)
        plsc.subcore_barrier()

        for k in range(RPT // CE):
            r0 = sid * RPT + k * CE
            pltpu.sync_copy(acc_sh.at[pl.ds(r0, CE)], rows_v)
            pltpu.sync_copy(rows_v, out.at[cid, pl.ds(r0, CE)])

    return pl.kernel(
        body,
        out_type=jax.ShapeDtypeStruct((NC, HROWS, feat), jnp.float32),
        mesh=mesh,
        scratch_types=scratch,
    )


def _make_deg_pass():
    mesh = plsc.VectorSubcoreMesh(core_axis_name="c", subcore_axis_name="s")
    scratch = [
        pltpu.VMEM((CPT, CE), jnp.int32),
        pltpu.VMEM((CE,), jnp.int32),
        pltpu.VMEM((CE, LANES), jnp.float32),
        pltpu.VMEM_SHARED((HROWS, LANES), jnp.float32),
    ]

    def body(dst, out, dst_v, idx_v, buf_v, acc_sh):
        cid = lax.axis_index("c")
        sid = lax.axis_index("s")
        pltpu.sync_copy(dst.at[sid], dst_v)

        zeros = jnp.zeros((LANES,), jnp.float32)

        def zrow(i, _):
            buf_v[i, pl.ds(0, LANES)] = zeros
            return 0

        lax.fori_loop(0, CE, zrow, 0)
        for k in range(RPT // CE):
            pltpu.sync_copy(buf_v, acc_sh.at[pl.ds(sid * RPT + k * CE, CE)])
        plsc.subcore_barrier()

        ones = jnp.full((LANES,), 1.0, jnp.float32)

        def orow(i, _):
            buf_v[i, pl.ds(0, LANES)] = ones
            return 0

        lax.fori_loop(0, CE, orow, 0)
        base = cid * HALF

        def chunk(c, _):
            for j in range(CE // LANES):
                sl = pl.ds(j * LANES, LANES)
                d = dst_v[c, sl] - base
                ok = (d >= 0) & (d < HALF)
                idx_v[sl] = jnp.where(ok, d, HALF)
            pltpu.sync_copy(buf_v, acc_sh.at[idx_v], add=True)
            return 0

        lax.fori_loop(0, CPT, chunk, 0)
        plsc.subcore_barrier()

        for k in range(RPT // CE):
            r0 = sid * RPT + k * CE
            pltpu.sync_copy(acc_sh.at[pl.ds(r0, CE)], buf_v)
            pltpu.sync_copy(buf_v, out.at[cid, pl.ds(r0, CE)])

    return pl.kernel(
        body,
        out_type=jax.ShapeDtypeStruct((NC, HROWS, LANES), jnp.float32),
        mesh=mesh,
        scratch_types=scratch,
    )


def _contract0(a, b):
    return lax.dot_general(a, b, (((0,), (0,)), ((), ())))


def _graph_norm(x, oh, gamma, beta, alpha):
    ones = jnp.ones((x.shape[0], 1), jnp.float32)
    cnt = jnp.maximum(_contract0(oh, ones), 1.0)
    mean = _contract0(oh, x) / cnt
    xc = x - alpha * jnp.dot(oh, mean)
    var = _contract0(oh, xc * xc) / cnt
    vb = jnp.dot(oh, var)
    return gamma * xc * lax.rsqrt(vb + EPS) + beta


def _onehot(batch_col):
    g = lax.broadcasted_iota(jnp.int32, (N, NG), 1)
    return (batch_col == g).astype(jnp.float32)


def _cat(a0, a1):
    return jnp.concatenate([a0, a1], axis=0)


def _encoder_body(point, cond, tim, batch, W1, b1, g1, be1, a1, W2, b2,
                  h_out):
    h1 = (jnp.dot(point[...], W1[0:3, :])
          + jnp.dot(cond[...], W1[3:7, :])
          + jnp.dot(tim[...], W1[7:8, :]) + b1[...])
    oh = _onehot(batch[...])
    y = _graph_norm(h1, oh, g1[...], be1[...], a1[...])
    h_out[...] = jnp.dot(jnp.maximum(y, 0.0), W2[...]) + b2[...]


def _layer0_body(h, Wn, hw_out):
    hw_out[...] = jnp.dot(h[...], Wn[...])


def _layer_body(h, a0, a1, Wn, h_out, hw_out):
    hn = jnp.maximum(h[...] + _cat(a0[...], a1[...]), 0.0)
    h_out[...] = hn
    hw_out[...] = jnp.dot(hn, Wn[...])


def _ae_body(attr, eeW, eeb, We, bl, ae_out):
    i = pl.program_id(0)
    at = jnp.dot(attr[...], eeW[...]) + eeb[...]
    ae = jnp.dot(at, We[...]) + bl[...]
    rows = i * BE + lax.broadcasted_iota(jnp.int32, (BE, 128), 0)
    ae_out[...] = jnp.where(rows < E, ae, NEG)


def _dec1_pre_body(h, a0, a1, d0, d1, Wd, y_out, dinv_out):
    hf = jnp.maximum(h[...] + _cat(a0[...], a1[...]), 0.0)
    dinv = lax.rsqrt(_cat(d0[...], d1[...]) + 1.0)
    dinv_out[...] = dinv
    y_out[...] = jnp.dot(hf, Wd[...]) * dinv


def _dec1_post_body(y, c0, c1, dinv, b1, batch, g1, be1, a1, Wd2, y2_out):
    dv = dinv[...]
    x = dv * (y[...] + _cat(c0[...], c1[...])) + b1[...]
    oh = _onehot(batch[...])
    x = jnp.maximum(_graph_norm(x, oh, g1[...], be1[...], a1[...]), 0.0)
    xw = jnp.dot(x, Wd2[...]) * dv
    y2_out[...] = jnp.concatenate(
        [xw, jnp.zeros((N, 64), jnp.float32)], axis=1)


def _final_body(y2, c0, c1, dinv, b2, batch, g2, be2, a2, fcW, fcb, out):
    cc = _cat(c0[...], c1[...])
    x = dinv[...] * (y2[:, 0:64] + cc[:, 0:64]) + b2[...]
    oh = _onehot(batch[...])
    x = jnp.maximum(_graph_norm(x, oh, g2[...], be2[...], a2[...]), 0.0)
    out[...] = jnp.dot(x, fcW[...]) + fcb[...]


def _tc_call(body, out_shapes):
    return pl.pallas_call(body, out_shape=out_shapes)


def kernel(point, condition, index, attr, batch, time, enc_W1, enc_b1,
           enc_g1, enc_be1, enc_a1, enc_W2, enc_b2, ee_W, ee_b, gcn_Wn,
           gcn_We, gcn_bl, dec_W1, dec_b1, dec_g1, dec_be1, dec_a1, dec_W2,
           dec_b2, dec_g2, dec_be2, dec_a2, fc_W, fc_b):
    f32 = jnp.float32
    src = index[0].astype(jnp.int32)
    dst = index[1].astype(jnp.int32)

    src_p = jnp.pad(src, (0, E_PAD - E)).reshape(NS, CPT, CE)
    dst_p = jnp.pad(dst, (0, E_PAD - E),
                    constant_values=DPAD).reshape(NS, CPT, CE)
    attr_p = jnp.pad(attr, ((0, E_PAD - E), (0, 0)))

    batch_c = batch.astype(jnp.int32).reshape(N, 1)
    row1 = lambda v: v.reshape(1, -1)

    edge_relu = _make_edge_pass(True)
    edge_gs = _make_edge_pass(False)
    deg_pass = _make_deg_pass()

    def parts(aggs):
        return aggs[0, :HALF], aggs[1, :N - HALF]

    h = _tc_call(_encoder_body, jax.ShapeDtypeStruct((N, 128), f32))(
        point, condition, time, batch_c, enc_W1, row1(enc_b1),
        row1(enc_g1), row1(enc_be1), row1(enc_a1), enc_W2, row1(enc_b2))

    degs = deg_pass(dst_p)
    d0 = degs[0, :HALF, 0:1]
    d1 = degs[1, :N - HALF, 0:1]

    ae_call = pl.pallas_call(
        _ae_body,
        grid=(E_PAD // BE,),
        in_specs=[
            pl.BlockSpec((BE, 16), lambda i: (i, 0)),
            pl.BlockSpec((16, 16), lambda i: (0, 0)),
            pl.BlockSpec((1, 16), lambda i: (0, 0)),
            pl.BlockSpec((16, 128), lambda i: (0, 0)),
            pl.BlockSpec((1, 128), lambda i: (0, 0)),
        ],
        out_specs=pl.BlockSpec((BE, 128), lambda i: (i, 0)),
        out_shape=jax.ShapeDtypeStruct((E_PAD, 128), f32),
    )

    agg = None
    for l in range(L):
        ae = ae_call(attr_p, ee_W, row1(ee_b), gcn_We[l], row1(gcn_bl[l]))
        if l == 0:
            hw = _tc_call(_layer0_body,
                          jax.ShapeDtypeStruct((N, 128), f32))(h, gcn_Wn[0])
        else:
            a0, a1 = parts(agg)
            h, hw = _tc_call(
                _layer_body,
                (jax.ShapeDtypeStruct((N, 128), f32),
                 jax.ShapeDtypeStruct((N, 128), f32)))(h, a0, a1, gcn_Wn[l])
        agg = edge_relu(hw, ae, src_p, dst_p)

    a0, a1 = parts(agg)
    y, dinv = _tc_call(
        _dec1_pre_body,
        (jax.ShapeDtypeStruct((N, 128), f32),
         jax.ShapeDtypeStruct((N, 1), f32)))(h, a0, a1, d0, d1, dec_W1)
    cagg = edge_gs(y, y, src_p, dst_p)

    c0, c1 = parts(cagg)
    y2 = _tc_call(_dec1_post_body, jax.ShapeDtypeStruct((N, 128), f32))(
        y, c0, c1, dinv, row1(dec_b1), batch_c, row1(dec_g1),
        row1(dec_be1), row1(dec_a1), dec_W2)
    cagg2 = edge_gs(y2, y2, src_p, dst_p)

    c0, c1 = parts(cagg2)
    out = _tc_call(_final_body, jax.ShapeDtypeStruct((N, 2), f32))(
        y2, c0, c1, dinv, row1(dec_b2), batch_c, row1(dec_g2),
        row1(dec_be2), row1(dec_a2), fc_W, row1(fc_b))
    return out


# speedup vs baseline: 1.5915x; 1.0145x over previous
import jax
import jax.numpy as jnp
from jax import lax
from jax.experimental import pallas as pl
from jax.experimental.pallas import tpu as pltpu
from jax.experimental.pallas import tpu_sc as plsc

N = 10000
E = 320000
NG = 16
L = 4
EPS = 1e-5
NEG = -1e30

NC = 2
NS = 16
LANES = 16
CE = 128
CPT = 157
EPT = CPT * CE
E_PAD = NS * EPT
HALF = 5120
HROWS = 6144
RPT = HROWS // NS
DPAD = 99999
BE = 2048


def _make_edge_pass(with_ae):
    feat = 128
    mesh = plsc.VectorSubcoreMesh(core_axis_name="c", subcore_axis_name="s")
    scratch = [
        pltpu.VMEM((CPT, CE), jnp.int32),
        pltpu.VMEM((CPT, CE), jnp.int32),
        pltpu.VMEM((CE,), jnp.int32),
        pltpu.VMEM((CE, feat), jnp.float32),
        pltpu.VMEM((CE, feat), jnp.float32),
        pltpu.VMEM_SHARED((HROWS, feat), jnp.float32),
        pltpu.SemaphoreType.DMA,
        pltpu.SemaphoreType.DMA,
    ]

    def body(table, ae, src, dst, out, src_v, dst_v, idx_v, rows_v, ae_v,
             acc_sh, sem, sem2):
        cid = lax.axis_index("c")
        sid = lax.axis_index("s")

        pltpu.sync_copy(src.at[sid], src_v)
        pltpu.sync_copy(dst.at[sid], dst_v)

        zeros = jnp.zeros((LANES,), jnp.float32)

        def zrow(i, _):
            for j in range(feat // LANES):
                rows_v[i, pl.ds(j * LANES, LANES)] = zeros
            return 0

        lax.fori_loop(0, CE, zrow, 0)
        for k in range(RPT // CE):
            pltpu.sync_copy(rows_v, acc_sh.at[pl.ds(sid * RPT + k * CE, CE)])
        plsc.subcore_barrier()

        base = cid * HALF

        def chunk(c, _):
            g = pltpu.async_copy(table.at[src_v.at[c]], rows_v, sem)
            if with_ae:
                a = pltpu.async_copy(
                    ae.at[pl.ds((sid * CPT + c) * CE, CE)], ae_v, sem2)
            for j in range(CE // LANES):
                sl = pl.ds(j * LANES, LANES)
                d = dst_v[c, sl] - base
                ok = (d >= 0) & (d < HALF)
                idx_v[sl] = jnp.where(ok, d, HALF)
            g.wait()
            if with_ae:
                a.wait()

                def erow(i, _):
                    for j in range(feat // LANES):
                        sl = pl.ds(j * LANES, LANES)
                        rows_v[i, sl] = jnp.maximum(
                            rows_v[i, sl] + ae_v[i, sl], 0.0)
                    return 0

                lax.fori_loop(0, CE, erow, 0)
            pltpu.sync_copy(rows_v, acc_sh.at[idx_v], add=True)
            return 0

        lax.fori_loop(0, CPT, chunk, 0)
        plsc.subcore_barrier()

        for k in range(RPT // CE):
            r0 = sid * RPT + k * CE
            pltpu.sync_copy(acc_sh.at[pl.ds(r0, CE)], rows_v)
            pltpu.sync_copy(rows_v, out.at[cid, pl.ds(r0, CE)])

    return pl.kernel(
        body,
        out_type=jax.ShapeDtypeStruct((NC, HROWS, feat), jnp.float32),
        mesh=mesh,
        scratch_types=scratch,
    )


def _make_edge_pass_pipe():
    feat = 128
    mesh = plsc.VectorSubcoreMesh(core_axis_name="c", subcore_axis_name="s")
    scratch = [
        pltpu.VMEM((CPT, CE), jnp.int32),
        pltpu.VMEM((CPT, CE), jnp.int32),
        pltpu.VMEM((CE,), jnp.int32),
        pltpu.VMEM((CE,), jnp.int32),
        pltpu.VMEM((CE, feat), jnp.float32),
        pltpu.VMEM((CE, feat), jnp.float32),
        [pltpu.SemaphoreType.DMA, pltpu.SemaphoreType.DMA],
        [pltpu.SemaphoreType.DMA, pltpu.SemaphoreType.DMA],
        pltpu.VMEM_SHARED((HROWS, feat), jnp.float32),
    ]

    def body(table, src, dst, out, src_v, dst_v, idx0, idx1, rows0, rows1,
             semg, sems, acc_sh):
        idx_b = (idx0, idx1)
        rows_b = (rows0, rows1)
        cid = lax.axis_index("c")
        sid = lax.axis_index("s")

        pltpu.sync_copy(src.at[sid], src_v)
        pltpu.sync_copy(dst.at[sid], dst_v)

        zeros = jnp.zeros((LANES,), jnp.float32)

        def zrow(i, _):
            for j in range(feat // LANES):
                rows0[i, pl.ds(j * LANES, LANES)] = zeros
            return 0

        lax.fori_loop(0, CE, zrow, 0)
        for k in range(RPT // CE):
            pltpu.sync_copy(rows0, acc_sh.at[pl.ds(sid * RPT + k * CE, CE)])
        plsc.subcore_barrier()

        base = cid * HALF

        def remap(c, b):
            for j in range(CE // LANES):
                sl = pl.ds(j * LANES, LANES)
                d = dst_v[c, sl] - base
                ok = (d >= 0) & (d < HALF)
                idx_b[b][sl] = jnp.where(ok, d, HALF)

        def gather(c, b):
            pltpu.async_copy(table.at[src_v.at[c]], rows_b[b], semg[b])

        def wait_gather(c, b):
            pltpu.make_async_copy(table.at[src_v.at[c]], rows_b[b],
                                  semg[b]).wait()

        def scatter(b):
            pltpu.async_copy(rows_b[b], acc_sh.at[idx_b[b]], sems[b],
                             add=True)

        def wait_scatter(b):
            pltpu.make_async_copy(rows_b[b], acc_sh.at[idx_b[b]],
                                  sems[b]).wait()

        gather(0, 0)

        def outer(co, _):
            c0 = 2 * co

            @pl.when(co > 0)
            def _():
                wait_scatter(1)
            gather(c0 + 1, 1)
            wait_gather(c0, 0)
            remap(c0, 0)
            scatter(0)

            wait_gather(c0 + 1, 1)
            remap(c0 + 1, 1)
            scatter(1)
            wait_scatter(0)
            gather(c0 + 2, 0)
            return 0

        lax.fori_loop(0, (CPT - 1) // 2, outer, 0)
        c_last = CPT - 1
        wait_scatter(1)
        wait_gather(c_last, 0)
        remap(c_last, 0)
        scatter(0)
        wait_scatter(0)
        plsc.subcore_barrier()

        for k in range(RPT // CE):
            r0 = sid * RPT + k * CE
            pltpu.sync_copy(acc_sh.at[pl.ds(r0, CE)], rows0)
            pltpu.sync_copy(rows0, out.at[cid, pl.ds(r0, CE)])

    return pl.kernel(
        body,
        out_type=jax.ShapeDtypeStruct((NC, HROWS, feat), jnp.float32),
        mesh=mesh,
        scratch_types=scratch,
    )


def _make_deg_pass():
    mesh = plsc.VectorSubcoreMesh(core_axis_name="c", subcore_axis_name="s")
    scratch = [
        pltpu.VMEM((CPT, CE), jnp.int32),
        pltpu.VMEM((CE,), jnp.int32),
        pltpu.VMEM((CE, LANES), jnp.float32),
        pltpu.VMEM_SHARED((HROWS, LANES), jnp.float32),
    ]

    def body(dst, out, dst_v, idx_v, buf_v, acc_sh):
        cid = lax.axis_index("c")
        sid = lax.axis_index("s")
        pltpu.sync_copy(dst.at[sid], dst_v)

        zeros = jnp.zeros((LANES,), jnp.float32)

        def zrow(i, _):
            buf_v[i, pl.ds(0, LANES)] = zeros
            return 0

        lax.fori_loop(0, CE, zrow, 0)
        for k in range(RPT // CE):
            pltpu.sync_copy(buf_v, acc_sh.at[pl.ds(sid * RPT + k * CE, CE)])
        plsc.subcore_barrier()

        ones = jnp.full((LANES,), 1.0, jnp.float32)

        def orow(i, _):
            buf_v[i, pl.ds(0, LANES)] = ones
            return 0

        lax.fori_loop(0, CE, orow, 0)
        base = cid * HALF

        def chunk(c, _):
            for j in range(CE // LANES):
                sl = pl.ds(j * LANES, LANES)
                d = dst_v[c, sl] - base
                ok = (d >= 0) & (d < HALF)
                idx_v[sl] = jnp.where(ok, d, HALF)
            pltpu.sync_copy(buf_v, acc_sh.at[idx_v], add=True)
            return 0

        lax.fori_loop(0, CPT, chunk, 0)
        plsc.subcore_barrier()

        for k in range(RPT // CE):
            r0 = sid * RPT + k * CE
            pltpu.sync_copy(acc_sh.at[pl.ds(r0, CE)], buf_v)
            pltpu.sync_copy(buf_v, out.at[cid, pl.ds(r0, CE)])

    return pl.kernel(
        body,
        out_type=jax.ShapeDtypeStruct((NC, HROWS, LANES), jnp.float32),
        mesh=mesh,
        scratch_types=scratch,
    )


def _contract0(a, b):
    return lax.dot_general(a, b, (((0,), (0,)), ((), ())))


def _graph_norm(x, oh, gamma, beta, alpha):
    ones = jnp.ones((x.shape[0], 1), jnp.float32)
    cnt = jnp.maximum(_contract0(oh, ones), 1.0)
    mean = _contract0(oh, x) / cnt
    xc = x - alpha * jnp.dot(oh, mean)
    var = _contract0(oh, xc * xc) / cnt
    vb = jnp.dot(oh, var)
    return gamma * xc * lax.rsqrt(vb + EPS) + beta


def _onehot(batch_col):
    g = lax.broadcasted_iota(jnp.int32, (N, NG), 1)
    return (batch_col == g).astype(jnp.float32)


def _cat(a0, a1):
    return jnp.concatenate([a0, a1], axis=0)


def _encoder_body(point, cond, tim, batch, W1, b1, g1, be1, a1, W2, b2,
                  h_out):
    h1 = (jnp.dot(point[...], W1[0:3, :])
          + jnp.dot(cond[...], W1[3:7, :])
          + jnp.dot(tim[...], W1[7:8, :]) + b1[...])
    oh = _onehot(batch[...])
    y = _graph_norm(h1, oh, g1[...], be1[...], a1[...])
    h_out[...] = jnp.dot(jnp.maximum(y, 0.0), W2[...]) + b2[...]


def _layer0_body(h, Wn, hw_out):
    hw_out[...] = jnp.dot(h[...], Wn[...])


def _layer_body(h, a0, a1, Wn, h_out, hw_out):
    hn = jnp.maximum(h[...] + _cat(a0[...], a1[...]), 0.0)
    h_out[...] = hn
    hw_out[...] = jnp.dot(hn, Wn[...])


def _ae_body(attr, eeW, eeb, We, bl, ae_out):
    i = pl.program_id(0)
    at = jnp.dot(attr[...], eeW[...]) + eeb[...]
    ae = jnp.dot(at, We[...]) + bl[...]
    rows = i * BE + lax.broadcasted_iota(jnp.int32, (BE, 128), 0)
    ae_out[...] = jnp.where(rows < E, ae, NEG)


def _dec1_pre_body(h, a0, a1, d0, d1, Wd, y_out, dinv_out):
    hf = jnp.maximum(h[...] + _cat(a0[...], a1[...]), 0.0)
    dinv = lax.rsqrt(_cat(d0[...], d1[...]) + 1.0)
    dinv_out[...] = dinv
    y_out[...] = jnp.dot(hf, Wd[...]) * dinv


def _dec1_post_body(y, c0, c1, dinv, b1, batch, g1, be1, a1, Wd2, y2_out):
    dv = dinv[...]
    x = dv * (y[...] + _cat(c0[...], c1[...])) + b1[...]
    oh = _onehot(batch[...])
    x = jnp.maximum(_graph_norm(x, oh, g1[...], be1[...], a1[...]), 0.0)
    xw = jnp.dot(x, Wd2[...]) * dv
    y2_out[...] = jnp.concatenate(
        [xw, jnp.zeros((N, 64), jnp.float32)], axis=1)


def _final_body(y2, c0, c1, dinv, b2, batch, g2, be2, a2, fcW, fcb, out):
    cc = _cat(c0[...], c1[...])
    x = dinv[...] * (y2[:, 0:64] + cc[:, 0:64]) + b2[...]
    oh = _onehot(batch[...])
    x = jnp.maximum(_graph_norm(x, oh, g2[...], be2[...], a2[...]), 0.0)
    out[...] = jnp.dot(x, fcW[...]) + fcb[...]


def _tc_call(body, out_shapes):
    return pl.pallas_call(body, out_shape=out_shapes)


def kernel(point, condition, index, attr, batch, time, enc_W1, enc_b1,
           enc_g1, enc_be1, enc_a1, enc_W2, enc_b2, ee_W, ee_b, gcn_Wn,
           gcn_We, gcn_bl, dec_W1, dec_b1, dec_g1, dec_be1, dec_a1, dec_W2,
           dec_b2, dec_g2, dec_be2, dec_a2, fc_W, fc_b):
    f32 = jnp.float32
    src = index[0].astype(jnp.int32)
    dst = index[1].astype(jnp.int32)

    src_p = jnp.pad(src, (0, E_PAD - E)).reshape(NS, CPT, CE)
    dst_p = jnp.pad(dst, (0, E_PAD - E),
                    constant_values=DPAD).reshape(NS, CPT, CE)
    attr_p = jnp.pad(attr, ((0, E_PAD - E), (0, 0)))

    batch_c = batch.astype(jnp.int32).reshape(N, 1)
    row1 = lambda v: v.reshape(1, -1)

    edge_relu = _make_edge_pass(True)
    edge_gs = _make_edge_pass_pipe()
    deg_pass = _make_deg_pass()

    def parts(aggs):
        return aggs[0, :HALF], aggs[1, :N - HALF]

    h = _tc_call(_encoder_body, jax.ShapeDtypeStruct((N, 128), f32))(
        point, condition, time, batch_c, enc_W1, row1(enc_b1),
        row1(enc_g1), row1(enc_be1), row1(enc_a1), enc_W2, row1(enc_b2))

    degs = deg_pass(dst_p)
    d0 = degs[0, :HALF, 0:1]
    d1 = degs[1, :N - HALF, 0:1]

    ae_call = pl.pallas_call(
        _ae_body,
        grid=(E_PAD // BE,),
        in_specs=[
            pl.BlockSpec((BE, 16), lambda i: (i, 0)),
            pl.BlockSpec((16, 16), lambda i: (0, 0)),
            pl.BlockSpec((1, 16), lambda i: (0, 0)),
            pl.BlockSpec((16, 128), lambda i: (0, 0)),
            pl.BlockSpec((1, 128), lambda i: (0, 0)),
        ],
        out_specs=pl.BlockSpec((BE, 128), lambda i: (i, 0)),
        out_shape=jax.ShapeDtypeStruct((E_PAD, 128), f32),
    )

    agg = None
    for l in range(L):
        ae = ae_call(attr_p, ee_W, row1(ee_b), gcn_We[l], row1(gcn_bl[l]))
        if l == 0:
            hw = _tc_call(_layer0_body,
                          jax.ShapeDtypeStruct((N, 128), f32))(h, gcn_Wn[0])
        else:
            a0, a1 = parts(agg)
            h, hw = _tc_call(
                _layer_body,
                (jax.ShapeDtypeStruct((N, 128), f32),
                 jax.ShapeDtypeStruct((N, 128), f32)))(h, a0, a1, gcn_Wn[l])
        agg = edge_relu(hw, ae, src_p, dst_p)

    a0, a1 = parts(agg)
    y, dinv = _tc_call(
        _dec1_pre_body,
        (jax.ShapeDtypeStruct((N, 128), f32),
         jax.ShapeDtypeStruct((N, 1), f32)))(h, a0, a1, d0, d1, dec_W1)
    cagg = edge_gs(y, src_p, dst_p)

    c0, c1 = parts(cagg)
    y2 = _tc_call(_dec1_post_body, jax.ShapeDtypeStruct((N, 128), f32))(
        y, c0, c1, dinv, row1(dec_b1), batch_c, row1(dec_g1),
        row1(dec_be1), row1(dec_a1), dec_W2)
    cagg2 = edge_gs(y2, src_p, dst_p)

    c0, c1 = parts(cagg2)
    out = _tc_call(_final_body, jax.ShapeDtypeStruct((N, 2), f32))(
        y2, c0, c1, dinv, row1(dec_b2), batch_c, row1(dec_g2),
        row1(dec_be2), row1(dec_a2), fc_W, row1(fc_b))
    return out
